# Initial kernel scaffold; baseline (speedup 1.0000x reference)
import jax
import jax.numpy as jnp
from jax.experimental import pallas as pl


def kernel(users, pos, neg, edge_index, edge_weight, user_emb, item_emb):
    raise NotImplementedError("write your pallas kernel here")


# speedup vs baseline: 3.9200x; 3.9200x over previous
import functools

import jax
import jax.numpy as jnp
from jax import lax
from jax.experimental import pallas as pl
from jax.experimental.pallas import tpu as pltpu
from jax.experimental.pallas import tpu_sc as plsc

N_USERS = 50000
M_ITEMS = 50000
D = 32
N = N_USERS + M_ITEMS
E = 1600000
B = 4096

NC = 2
NS = 16
HALF = N // NC
ROWS_PT = 3128
ROWS_MAIN = 3072
DUMP = HALF

CHUNK = 128
SUPER = 4096
NCH = SUPER // CHUNK
E_PAD = 1638400
EPT = E_PAD // NS
NSUP = EPT // SUPER

SIDX = 3 * B
IPW = SIDX // (NC * NS)
NSC = IPW // CHUNK

_mesh = plsc.VectorSubcoreMesh(core_axis_name="c", subcore_axis_name="s")


def _layer_body(src_hbm, dst_hbm, w_hbm, tbl_hbm, out_hbm,
                src_v, dst_v, w_v, dloc_v, rows_v, acc_sh, sem):
    c = lax.axis_index("c")
    s = lax.axis_index("s")
    base_row = c * HALF
    r0 = s * ROWS_PT

    zero16 = jnp.zeros((16,), jnp.float32)

    def zfill(j, _):
        rows_v[j, 0:16] = zero16
        rows_v[j, 16:32] = zero16
        return 0
    lax.fori_loop(0, CHUNK, zfill, 0)

    def zbody(k, _):
        pltpu.sync_copy(rows_v, acc_sh.at[pl.ds(r0 + k * CHUNK, CHUNK)])
        return 0
    lax.fori_loop(0, ROWS_MAIN // CHUNK, zbody, 0)

    @pl.when(s < NS - 1)
    def _():
        pltpu.sync_copy(rows_v.at[pl.ds(0, 56)],
                        acc_sh.at[pl.ds(r0 + ROWS_MAIN, 56)])

    @pl.when(s == NS - 1)
    def _():
        pltpu.sync_copy(rows_v.at[pl.ds(0, 8)],
                        acc_sh.at[pl.ds(r0 + ROWS_MAIN, 8)])

    plsc.subcore_barrier()

    ebase0 = s * EPT

    def super_body(g, _):
        eb = ebase0 + g * SUPER
        pltpu.sync_copy(src_hbm.at[pl.ds(eb, SUPER)], src_v)
        pltpu.sync_copy(dst_hbm.at[pl.ds(eb, SUPER)], dst_v)
        pltpu.sync_copy(w_hbm.at[pl.ds(eb, SUPER)], w_v)

        def chunk_body(k, _):
            cb = k * CHUNK
            pltpu.async_copy(tbl_hbm.at[src_v.at[pl.ds(cb, CHUNK)]],
                             rows_v, sem).wait()

            def grp(q, _):
                d = dst_v[pl.ds(cb + q * 16, 16)]
                inr = (d >= base_row) & (d < base_row + HALF)
                dloc_v[pl.ds(q * 16, 16)] = jnp.where(inr, d - base_row, DUMP)
                return 0
            lax.fori_loop(0, CHUNK // 16, grp, 0)

            def edge(j, _):
                widx = jnp.full((16,), cb + j, jnp.int32)
                ws = plsc.load_gather(w_v, [widx])
                rows_v[j, 0:16] = rows_v[j, 0:16] * ws
                rows_v[j, 16:32] = rows_v[j, 16:32] * ws
                return 0
            lax.fori_loop(0, CHUNK, edge, 0)

            pltpu.sync_copy(rows_v, acc_sh.at[dloc_v], add=True)
            return 0
        lax.fori_loop(0, NCH, chunk_body, 0)
        return 0
    lax.fori_loop(0, NSUP, super_body, 0)

    plsc.subcore_barrier()
    pltpu.sync_copy(acc_sh.at[pl.ds(r0, ROWS_MAIN)],
                    out_hbm.at[pl.ds(base_row + r0, ROWS_MAIN)])

    @pl.when(s < NS - 1)
    def _():
        pltpu.sync_copy(acc_sh.at[pl.ds(r0 + ROWS_MAIN, 56)],
                        out_hbm.at[pl.ds(base_row + r0 + ROWS_MAIN, 56)])

    @pl.when(s == NS - 1)
    def _():
        pltpu.sync_copy(acc_sh.at[pl.ds(r0 + ROWS_MAIN, 8)],
                        out_hbm.at[pl.ds(base_row + r0 + ROWS_MAIN, 8)])


_sc_params = pltpu.CompilerParams(needs_layout_passes=False,
                                 use_tc_tiling_on_sc=False)

_layer = functools.partial(
    pl.kernel,
    out_type=jax.ShapeDtypeStruct((N, D), jnp.float32),
    mesh=_mesh,
    compiler_params=_sc_params,
    scratch_types=[
        pltpu.VMEM((SUPER,), jnp.int32),
        pltpu.VMEM((SUPER,), jnp.int32),
        pltpu.VMEM((SUPER,), jnp.float32),
        pltpu.VMEM((CHUNK,), jnp.int32),
        pltpu.VMEM((CHUNK, D), jnp.float32),
        pltpu.VMEM_SHARED((HALF + 8, D), jnp.float32),
        pltpu.SemaphoreType.DMA,
    ],
)(_layer_body)


def _sampler_body(e0, e1, e2, e3, idx_hbm, out_sum, out_e0,
                  idx_v, identw_v, rows_a, rows_b, sum_sh, sem):
    c = lax.axis_index("c")
    s = lax.axis_index("s")
    w = s * NC + c
    base = w * IPW
    srow = s * CHUNK

    def ibody(q, _):
        identw_v[pl.ds(q * 16, 16)] = lax.iota(jnp.int32, 16) + (q * 16 + srow)
        return 0
    lax.fori_loop(0, CHUNK // 16, ibody, 0)

    pltpu.sync_copy(idx_hbm.at[pl.ds(base, IPW)], idx_v)

    def sub(k, _):
        sl = idx_v.at[pl.ds(k * CHUNK, CHUNK)]
        pltpu.async_copy(e0.at[sl], rows_a, sem).wait()
        pltpu.sync_copy(rows_a, out_e0.at[pl.ds(base + k * CHUNK, CHUNK)])
        pltpu.sync_copy(rows_a, sum_sh.at[pl.ds(srow, CHUNK)])
        pltpu.async_copy(e1.at[sl], rows_b, sem).wait()
        pltpu.sync_copy(rows_b, sum_sh.at[identw_v], add=True)
        pltpu.async_copy(e2.at[sl], rows_b, sem).wait()
        pltpu.sync_copy(rows_b, sum_sh.at[identw_v], add=True)
        pltpu.async_copy(e3.at[sl], rows_b, sem).wait()
        pltpu.sync_copy(rows_b, sum_sh.at[identw_v], add=True)
        pltpu.sync_copy(sum_sh.at[pl.ds(srow, CHUNK)],
                        out_sum.at[pl.ds(base + k * CHUNK, CHUNK)])
        return 0
    lax.fori_loop(0, NSC, sub, 0)


_sampler = functools.partial(
    pl.kernel,
    out_type=(jax.ShapeDtypeStruct((SIDX, D), jnp.float32),
              jax.ShapeDtypeStruct((SIDX, D), jnp.float32)),
    mesh=_mesh,
    compiler_params=_sc_params,
    scratch_types=[
        pltpu.VMEM((IPW,), jnp.int32),
        pltpu.VMEM((CHUNK,), jnp.int32),
        pltpu.VMEM((CHUNK, D), jnp.float32),
        pltpu.VMEM((CHUNK, D), jnp.float32),
        pltpu.VMEM_SHARED((NS * CHUNK, D), jnp.float32),
        pltpu.SemaphoreType.DMA,
    ],
)(_sampler_body)


def _loss_body(sum_ref, e0_ref, out_ref):
    su = sum_ref[0:B, :] * 0.25
    sp = sum_ref[B:2 * B, :] * 0.25
    sn = sum_ref[2 * B:3 * B, :] * 0.25
    x = jnp.sum(su * sp, axis=1) - jnp.sum(su * sn, axis=1)
    ls = jnp.minimum(x, 0.0) - jnp.log(1.0 + jnp.exp(-jnp.abs(x)))
    loss = -jnp.mean(ls)
    u0 = e0_ref[0:B, :]
    p0 = e0_ref[B:2 * B, :]
    n0 = e0_ref[2 * B:3 * B, :]
    reg = jnp.mean(jnp.sum(u0 * u0 + p0 * p0 + n0 * n0, axis=1))
    out_ref[0] = loss
    out_ref[1] = reg


def _loss_tc(sum_rows, e0_rows):
    return pl.pallas_call(
        _loss_body,
        out_shape=jax.ShapeDtypeStruct((2,), jnp.float32),
        in_specs=[pl.BlockSpec(memory_space=pltpu.VMEM),
                  pl.BlockSpec(memory_space=pltpu.VMEM)],
        out_specs=pl.BlockSpec(memory_space=pltpu.SMEM),
    )(sum_rows, e0_rows)


def kernel(users, pos, neg, edge_index, edge_weight, user_emb, item_emb):
    e0 = jnp.concatenate([user_emb, item_emb], axis=0)
    pad = E_PAD - E
    src = jnp.pad(edge_index[0], (0, pad))
    dst = jnp.pad(edge_index[1], (0, pad))
    w = jnp.pad(edge_weight, (0, pad))
    e1 = _layer(src, dst, w, e0)
    e2 = _layer(src, dst, w, e1)
    e3 = _layer(src, dst, w, e2)
    idx = jnp.concatenate([users, pos + N_USERS, neg + N_USERS])
    sum_rows, e0_rows = _sampler(e0, e1, e2, e3, idx)
    return _loss_tc(sum_rows, e0_rows)


# speedup vs baseline: 4.0303x; 1.0281x over previous
import functools

import jax
import jax.numpy as jnp
from jax import lax
from jax.experimental import pallas as pl
from jax.experimental.pallas import tpu as pltpu
from jax.experimental.pallas import tpu_sc as plsc

N_USERS = 50000
M_ITEMS = 50000
D = 32
N = N_USERS + M_ITEMS
E = 1600000
B = 4096

NC = 2
NS = 16
HALF = N // NC
ROWS_PT = 3128
ROWS_MAIN = 3072
DUMP = HALF

CHUNK = 128
SUPER = 4096
NCH = SUPER // CHUNK
E_PAD = 1638400
EPT = E_PAD // NS
NSUP = EPT // SUPER

SIDX = 3 * B
IPW = SIDX // (NC * NS)
NSC = IPW // CHUNK

_mesh = plsc.VectorSubcoreMesh(core_axis_name="c", subcore_axis_name="s")


def _layer_body(src_hbm, dst_hbm, w_hbm, tbl_hbm, out_hbm,
                src_v, dst_v, w_v, dloc_v, gsrc_v, rows_v, acc_sh, sem):
    c = lax.axis_index("c")
    s = lax.axis_index("s")
    base_row = c * HALF
    r0 = s * ROWS_PT

    zero16 = jnp.zeros((16,), jnp.float32)

    def zfill(j, _):
        rows_v[j, 0:16] = zero16
        rows_v[j, 16:32] = zero16
        return 0
    lax.fori_loop(0, CHUNK, zfill, 0)

    def zbody(k, _):
        pltpu.sync_copy(rows_v, acc_sh.at[pl.ds(r0 + k * CHUNK, CHUNK)])
        return 0
    lax.fori_loop(0, ROWS_MAIN // CHUNK, zbody, 0)

    @pl.when(s < NS - 1)
    def _():
        pltpu.sync_copy(rows_v.at[pl.ds(0, 56)],
                        acc_sh.at[pl.ds(r0 + ROWS_MAIN, 56)])

    @pl.when(s == NS - 1)
    def _():
        pltpu.sync_copy(rows_v.at[pl.ds(0, 8)],
                        acc_sh.at[pl.ds(r0 + ROWS_MAIN, 8)])

    plsc.subcore_barrier()

    ebase0 = s * EPT

    def super_body(g, _):
        eb = ebase0 + g * SUPER
        pltpu.sync_copy(src_hbm.at[pl.ds(eb, SUPER)], src_v)
        pltpu.sync_copy(dst_hbm.at[pl.ds(eb, SUPER)], dst_v)
        pltpu.sync_copy(w_hbm.at[pl.ds(eb, SUPER)], w_v)

        def chunk_body(k, _):
            cb = k * CHUNK

            def grp(q, _):
                d = dst_v[pl.ds(cb + q * 16, 16)]
                sv = src_v[pl.ds(cb + q * 16, 16)]
                inr = (d >= base_row) & (d < base_row + HALF)
                dloc_v[pl.ds(q * 16, 16)] = jnp.where(inr, d - base_row, -1)
                gsrc_v[pl.ds(q * 16, 16)] = jnp.where(inr, sv, -1)
                return 0
            lax.fori_loop(0, CHUNK // 16, grp, 0)

            pltpu.async_copy(tbl_hbm.at[plsc.Indices(gsrc_v, ignored_value=-1)],
                             rows_v, sem).wait()

            def edge(j, _):
                widx = jnp.full((16,), cb + j, jnp.int32)
                ws = plsc.load_gather(w_v, [widx])
                rows_v[j, 0:16] = rows_v[j, 0:16] * ws
                rows_v[j, 16:32] = rows_v[j, 16:32] * ws
                return 0
            lax.fori_loop(0, CHUNK, edge, 0)

            pltpu.sync_copy(rows_v, acc_sh.at[plsc.Indices(dloc_v, ignored_value=-1)], add=True)
            return 0
        lax.fori_loop(0, NCH, chunk_body, 0)
        return 0
    lax.fori_loop(0, NSUP, super_body, 0)

    plsc.subcore_barrier()
    pltpu.sync_copy(acc_sh.at[pl.ds(r0, ROWS_MAIN)],
                    out_hbm.at[pl.ds(base_row + r0, ROWS_MAIN)])

    @pl.when(s < NS - 1)
    def _():
        pltpu.sync_copy(acc_sh.at[pl.ds(r0 + ROWS_MAIN, 56)],
                        out_hbm.at[pl.ds(base_row + r0 + ROWS_MAIN, 56)])

    @pl.when(s == NS - 1)
    def _():
        pltpu.sync_copy(acc_sh.at[pl.ds(r0 + ROWS_MAIN, 8)],
                        out_hbm.at[pl.ds(base_row + r0 + ROWS_MAIN, 8)])


_sc_params = pltpu.CompilerParams(needs_layout_passes=False,
                                 use_tc_tiling_on_sc=False)

_layer = functools.partial(
    pl.kernel,
    out_type=jax.ShapeDtypeStruct((N, D), jnp.float32),
    mesh=_mesh,
    compiler_params=_sc_params,
    scratch_types=[
        pltpu.VMEM((SUPER,), jnp.int32),
        pltpu.VMEM((SUPER,), jnp.int32),
        pltpu.VMEM((SUPER,), jnp.float32),
        pltpu.VMEM((CHUNK,), jnp.int32),
        pltpu.VMEM((CHUNK,), jnp.int32),
        pltpu.VMEM((CHUNK, D), jnp.float32),
        pltpu.VMEM_SHARED((HALF + 8, D), jnp.float32),
        pltpu.SemaphoreType.DMA,
    ],
)(_layer_body)


def _sampler_body(e0, e1, e2, e3, idx_hbm, out_sum, out_e0,
                  idx_v, identw_v, rows_a, rows_b, sum_sh, sem):
    c = lax.axis_index("c")
    s = lax.axis_index("s")
    w = s * NC + c
    base = w * IPW
    srow = s * CHUNK

    def ibody(q, _):
        identw_v[pl.ds(q * 16, 16)] = lax.iota(jnp.int32, 16) + (q * 16 + srow)
        return 0
    lax.fori_loop(0, CHUNK // 16, ibody, 0)

    pltpu.sync_copy(idx_hbm.at[pl.ds(base, IPW)], idx_v)

    def sub(k, _):
        sl = idx_v.at[pl.ds(k * CHUNK, CHUNK)]
        pltpu.async_copy(e0.at[sl], rows_a, sem).wait()
        pltpu.sync_copy(rows_a, out_e0.at[pl.ds(base + k * CHUNK, CHUNK)])
        pltpu.sync_copy(rows_a, sum_sh.at[pl.ds(srow, CHUNK)])
        pltpu.async_copy(e1.at[sl], rows_b, sem).wait()
        pltpu.sync_copy(rows_b, sum_sh.at[identw_v], add=True)
        pltpu.async_copy(e2.at[sl], rows_b, sem).wait()
        pltpu.sync_copy(rows_b, sum_sh.at[identw_v], add=True)
        pltpu.async_copy(e3.at[sl], rows_b, sem).wait()
        pltpu.sync_copy(rows_b, sum_sh.at[identw_v], add=True)
        pltpu.sync_copy(sum_sh.at[pl.ds(srow, CHUNK)],
                        out_sum.at[pl.ds(base + k * CHUNK, CHUNK)])
        return 0
    lax.fori_loop(0, NSC, sub, 0)


_sampler = functools.partial(
    pl.kernel,
    out_type=(jax.ShapeDtypeStruct((SIDX, D), jnp.float32),
              jax.ShapeDtypeStruct((SIDX, D), jnp.float32)),
    mesh=_mesh,
    compiler_params=_sc_params,
    scratch_types=[
        pltpu.VMEM((IPW,), jnp.int32),
        pltpu.VMEM((CHUNK,), jnp.int32),
        pltpu.VMEM((CHUNK, D), jnp.float32),
        pltpu.VMEM((CHUNK, D), jnp.float32),
        pltpu.VMEM_SHARED((NS * CHUNK, D), jnp.float32),
        pltpu.SemaphoreType.DMA,
    ],
)(_sampler_body)


def _loss_body(sum_ref, e0_ref, out_ref):
    su = sum_ref[0:B, :] * 0.25
    sp = sum_ref[B:2 * B, :] * 0.25
    sn = sum_ref[2 * B:3 * B, :] * 0.25
    x = jnp.sum(su * sp, axis=1) - jnp.sum(su * sn, axis=1)
    ls = jnp.minimum(x, 0.0) - jnp.log(1.0 + jnp.exp(-jnp.abs(x)))
    loss = -jnp.mean(ls)
    u0 = e0_ref[0:B, :]
    p0 = e0_ref[B:2 * B, :]
    n0 = e0_ref[2 * B:3 * B, :]
    reg = jnp.mean(jnp.sum(u0 * u0 + p0 * p0 + n0 * n0, axis=1))
    out_ref[0] = loss
    out_ref[1] = reg


def _loss_tc(sum_rows, e0_rows):
    return pl.pallas_call(
        _loss_body,
        out_shape=jax.ShapeDtypeStruct((2,), jnp.float32),
        in_specs=[pl.BlockSpec(memory_space=pltpu.VMEM),
                  pl.BlockSpec(memory_space=pltpu.VMEM)],
        out_specs=pl.BlockSpec(memory_space=pltpu.SMEM),
    )(sum_rows, e0_rows)


def kernel(users, pos, neg, edge_index, edge_weight, user_emb, item_emb):
    e0 = jnp.concatenate([user_emb, item_emb], axis=0)
    pad = E_PAD - E
    src = jnp.pad(edge_index[0], (0, pad))
    dst = jnp.pad(edge_index[1], (0, pad))
    w = jnp.pad(edge_weight, (0, pad))
    e1 = _layer(src, dst, w, e0)
    e2 = _layer(src, dst, w, e1)
    e3 = _layer(src, dst, w, e2)
    idx = jnp.concatenate([users, pos + N_USERS, neg + N_USERS])
    sum_rows, e0_rows = _sampler(e0, e1, e2, e3, idx)
    return _loss_tc(sum_rows, e0_rows)


# speedup vs baseline: 5.8173x; 1.4434x over previous
import functools

import jax
import jax.numpy as jnp
from jax import lax
from jax.experimental import pallas as pl
from jax.experimental.pallas import tpu as pltpu
from jax.experimental.pallas import tpu_sc as plsc

N_USERS = 50000
M_ITEMS = 50000
D = 32
N = N_USERS + M_ITEMS
E = 1600000
B = 4096

NC = 2
NS = 16
HALF = N // NC
ROWS_PT = 3128
ROWS_MAIN = 3072
DUMP = HALF

CHUNK = 128
SUPER = 4096
NCH = SUPER // CHUNK
E_PAD = 1638400
EPT = E_PAD // NS
NSUP = EPT // SUPER

SIDX = 3 * B
IPW = SIDX // (NC * NS)
NSC = IPW // CHUNK

_mesh = plsc.VectorSubcoreMesh(core_axis_name="c", subcore_axis_name="s")


def _layer_body(src_hbm, dst_hbm, w_hbm, tbl_hbm, out_hbm,
                src_v, dst_v, w_v, dloc0, dloc1, gsrc0, gsrc1,
                rows0, rows1, acc_sh, gsem0, gsem1, ssem0, ssem1):
    c = lax.axis_index("c")
    s = lax.axis_index("s")
    base_row = c * HALF
    r0 = s * ROWS_PT

    zero16 = jnp.zeros((16,), jnp.float32)

    @plsc.parallel_loop(0, CHUNK, unroll=4)
    def _(j):
        rows0[j, 0:16] = zero16
        rows0[j, 16:32] = zero16

    def zbody(k, _):
        pltpu.sync_copy(rows0, acc_sh.at[pl.ds(r0 + k * CHUNK, CHUNK)])
        return 0
    lax.fori_loop(0, ROWS_MAIN // CHUNK, zbody, 0)

    @pl.when(s < NS - 1)
    def _():
        pltpu.sync_copy(rows0.at[pl.ds(0, 56)],
                        acc_sh.at[pl.ds(r0 + ROWS_MAIN, 56)])

    @pl.when(s == NS - 1)
    def _():
        pltpu.sync_copy(rows0.at[pl.ds(0, 8)],
                        acc_sh.at[pl.ds(r0 + ROWS_MAIN, 8)])

    plsc.subcore_barrier()

    ebase0 = s * EPT

    def prep(cb, dloc, gsrc):
        @plsc.parallel_loop(0, CHUNK // 16, unroll=2)
        def _(q):
            d = dst_v[pl.ds(cb + q * 16, 16)]
            sv = src_v[pl.ds(cb + q * 16, 16)]
            inr = (d >= base_row) & (d < base_row + HALF)
            dloc[pl.ds(q * 16, 16)] = jnp.where(inr, d - base_row, -1)
            gsrc[pl.ds(q * 16, 16)] = jnp.where(inr, sv, -1)

    def gather(gsrc, rows, gsem):
        pltpu.async_copy(tbl_hbm.at[plsc.Indices(gsrc, ignored_value=-1)],
                         rows, gsem)

    def wait_gather(gsrc, rows, gsem):
        pltpu.make_async_copy(tbl_hbm.at[plsc.Indices(gsrc, ignored_value=-1)],
                              rows, gsem).wait()

    def scatter(rows, dloc, ssem):
        pltpu.async_copy(rows, acc_sh.at[plsc.Indices(dloc, ignored_value=-1)],
                         ssem, add=True)

    def wait_scatter(rows, dloc, ssem):
        pltpu.make_async_copy(rows,
                              acc_sh.at[plsc.Indices(dloc, ignored_value=-1)],
                              ssem).wait()

    def mul(rows, wbase):
        @plsc.parallel_loop(0, CHUNK, unroll=4)
        def _(j):
            widx = jnp.full((16,), wbase + j, jnp.int32)
            ws = plsc.load_gather(w_v, [widx])
            rows[j, 0:16] = rows[j, 0:16] * ws
            rows[j, 16:32] = rows[j, 16:32] * ws

    def super_body(g, _):
        eb = ebase0 + g * SUPER
        pltpu.sync_copy(src_hbm.at[pl.ds(eb, SUPER)], src_v)
        pltpu.sync_copy(dst_hbm.at[pl.ds(eb, SUPER)], dst_v)
        pltpu.sync_copy(w_hbm.at[pl.ds(eb, SUPER)], w_v)

        prep(0, dloc0, gsrc0)
        gather(gsrc0, rows0, gsem0)

        def pair(p, _):
            wait_gather(gsrc0, rows0, gsem0)
            mul(rows0, 2 * p * CHUNK)

            @pl.when(p > 0)
            def _():
                wait_scatter(rows1, dloc1, ssem1)
            prep((2 * p + 1) * CHUNK, dloc1, gsrc1)
            gather(gsrc1, rows1, gsem1)
            scatter(rows0, dloc0, ssem0)

            wait_gather(gsrc1, rows1, gsem1)
            mul(rows1, (2 * p + 1) * CHUNK)

            @pl.when(p < NCH // 2 - 1)
            def _():
                wait_scatter(rows0, dloc0, ssem0)
                prep((2 * p + 2) * CHUNK, dloc0, gsrc0)
                gather(gsrc0, rows0, gsem0)
            scatter(rows1, dloc1, ssem1)
            return 0
        lax.fori_loop(0, NCH // 2, pair, 0)

        wait_scatter(rows0, dloc0, ssem0)
        wait_scatter(rows1, dloc1, ssem1)
        return 0
    lax.fori_loop(0, NSUP, super_body, 0)

    plsc.subcore_barrier()
    pltpu.sync_copy(acc_sh.at[pl.ds(r0, ROWS_MAIN)],
                    out_hbm.at[pl.ds(base_row + r0, ROWS_MAIN)])

    @pl.when(s < NS - 1)
    def _():
        pltpu.sync_copy(acc_sh.at[pl.ds(r0 + ROWS_MAIN, 56)],
                        out_hbm.at[pl.ds(base_row + r0 + ROWS_MAIN, 56)])

    @pl.when(s == NS - 1)
    def _():
        pltpu.sync_copy(acc_sh.at[pl.ds(r0 + ROWS_MAIN, 8)],
                        out_hbm.at[pl.ds(base_row + r0 + ROWS_MAIN, 8)])


_sc_params = pltpu.CompilerParams(needs_layout_passes=False,
                                 use_tc_tiling_on_sc=False)

_layer = functools.partial(
    pl.kernel,
    out_type=jax.ShapeDtypeStruct((N, D), jnp.float32),
    mesh=_mesh,
    compiler_params=_sc_params,
    scratch_types=[
        pltpu.VMEM((SUPER,), jnp.int32),
        pltpu.VMEM((SUPER,), jnp.int32),
        pltpu.VMEM((SUPER,), jnp.float32),
        pltpu.VMEM((CHUNK,), jnp.int32),
        pltpu.VMEM((CHUNK,), jnp.int32),
        pltpu.VMEM((CHUNK,), jnp.int32),
        pltpu.VMEM((CHUNK,), jnp.int32),
        pltpu.VMEM((CHUNK, D), jnp.float32),
        pltpu.VMEM((CHUNK, D), jnp.float32),
        pltpu.VMEM_SHARED((HALF + 8, D), jnp.float32),
        pltpu.SemaphoreType.DMA,
        pltpu.SemaphoreType.DMA,
        pltpu.SemaphoreType.DMA,
        pltpu.SemaphoreType.DMA,
    ],
)(_layer_body)


def _sampler_body(e0, e1, e2, e3, idx_hbm, out_sum, out_e0,
                  idx_v, identw_v, rows_a, rows_b, sum_sh, sem):
    c = lax.axis_index("c")
    s = lax.axis_index("s")
    w = s * NC + c
    base = w * IPW
    srow = s * CHUNK

    def ibody(q, _):
        identw_v[pl.ds(q * 16, 16)] = lax.iota(jnp.int32, 16) + (q * 16 + srow)
        return 0
    lax.fori_loop(0, CHUNK // 16, ibody, 0)

    pltpu.sync_copy(idx_hbm.at[pl.ds(base, IPW)], idx_v)

    def sub(k, _):
        sl = idx_v.at[pl.ds(k * CHUNK, CHUNK)]
        pltpu.async_copy(e0.at[sl], rows_a, sem).wait()
        pltpu.sync_copy(rows_a, out_e0.at[pl.ds(base + k * CHUNK, CHUNK)])
        pltpu.sync_copy(rows_a, sum_sh.at[pl.ds(srow, CHUNK)])
        pltpu.async_copy(e1.at[sl], rows_b, sem).wait()
        pltpu.sync_copy(rows_b, sum_sh.at[identw_v], add=True)
        pltpu.async_copy(e2.at[sl], rows_b, sem).wait()
        pltpu.sync_copy(rows_b, sum_sh.at[identw_v], add=True)
        pltpu.async_copy(e3.at[sl], rows_b, sem).wait()
        pltpu.sync_copy(rows_b, sum_sh.at[identw_v], add=True)
        pltpu.sync_copy(sum_sh.at[pl.ds(srow, CHUNK)],
                        out_sum.at[pl.ds(base + k * CHUNK, CHUNK)])
        return 0
    lax.fori_loop(0, NSC, sub, 0)


_sampler = functools.partial(
    pl.kernel,
    out_type=(jax.ShapeDtypeStruct((SIDX, D), jnp.float32),
              jax.ShapeDtypeStruct((SIDX, D), jnp.float32)),
    mesh=_mesh,
    compiler_params=_sc_params,
    scratch_types=[
        pltpu.VMEM((IPW,), jnp.int32),
        pltpu.VMEM((CHUNK,), jnp.int32),
        pltpu.VMEM((CHUNK, D), jnp.float32),
        pltpu.VMEM((CHUNK, D), jnp.float32),
        pltpu.VMEM_SHARED((NS * CHUNK, D), jnp.float32),
        pltpu.SemaphoreType.DMA,
    ],
)(_sampler_body)


def _loss_body(sum_ref, e0_ref, out_ref):
    su = sum_ref[0:B, :] * 0.25
    sp = sum_ref[B:2 * B, :] * 0.25
    sn = sum_ref[2 * B:3 * B, :] * 0.25
    x = jnp.sum(su * sp, axis=1) - jnp.sum(su * sn, axis=1)
    ls = jnp.minimum(x, 0.0) - jnp.log(1.0 + jnp.exp(-jnp.abs(x)))
    loss = -jnp.mean(ls)
    u0 = e0_ref[0:B, :]
    p0 = e0_ref[B:2 * B, :]
    n0 = e0_ref[2 * B:3 * B, :]
    reg = jnp.mean(jnp.sum(u0 * u0 + p0 * p0 + n0 * n0, axis=1))
    out_ref[0] = loss
    out_ref[1] = reg


def _loss_tc(sum_rows, e0_rows):
    return pl.pallas_call(
        _loss_body,
        out_shape=jax.ShapeDtypeStruct((2,), jnp.float32),
        in_specs=[pl.BlockSpec(memory_space=pltpu.VMEM),
                  pl.BlockSpec(memory_space=pltpu.VMEM)],
        out_specs=pl.BlockSpec(memory_space=pltpu.SMEM),
    )(sum_rows, e0_rows)


def kernel(users, pos, neg, edge_index, edge_weight, user_emb, item_emb):
    e0 = jnp.concatenate([user_emb, item_emb], axis=0)
    pad = E_PAD - E
    src = jnp.pad(edge_index[0], (0, pad))
    dst = jnp.pad(edge_index[1], (0, pad))
    w = jnp.pad(edge_weight, (0, pad))
    e1 = _layer(src, dst, w, e0)
    e2 = _layer(src, dst, w, e1)
    e3 = _layer(src, dst, w, e2)
    idx = jnp.concatenate([users, pos + N_USERS, neg + N_USERS])
    sum_rows, e0_rows = _sampler(e0, e1, e2, e3, idx)
    return _loss_tc(sum_rows, e0_rows)


# speedup vs baseline: 8.3279x; 1.4316x over previous
import functools

import jax
import jax.numpy as jnp
from jax import lax
from jax.experimental import pallas as pl
from jax.experimental.pallas import tpu as pltpu
from jax.experimental.pallas import tpu_sc as plsc

N_USERS = 50000
M_ITEMS = 50000
D = 32
N = N_USERS + M_ITEMS
E = 1600000
B = 4096

NC = 2
NS = 16
HALF = N // NC
ROWS_PT = 3128
ROWS_MAIN = 3072
DUMP = HALF

CHUNK = 128
SUPER = 4096
NCH = SUPER // CHUNK
E_PAD = 1638400
EPT = E_PAD // NS
NSUP = EPT // SUPER

SIDX = 3 * B
IPW = SIDX // (NC * NS)
NSC = IPW // CHUNK

_mesh = plsc.VectorSubcoreMesh(core_axis_name="c", subcore_axis_name="s")


def _layer_body(src_hbm, dst_hbm, w_hbm, tbl_hbm, out_hbm,
                src_v, dst_v, w_v, dloc0, dloc1, gsrc0, gsrc1,
                rows0, rows1, acc_sh, gsem0, gsem1, ssem0, ssem1):
    c = lax.axis_index("c")
    s = lax.axis_index("s")
    base_row = c * HALF
    r0 = s * ROWS_PT

    zero16 = jnp.zeros((16,), jnp.float32)

    @plsc.parallel_loop(0, CHUNK, unroll=4)
    def _(j):
        rows0[j, 0:16] = zero16
        rows0[j, 16:32] = zero16

    def zbody(k, _):
        pltpu.sync_copy(rows0, acc_sh.at[pl.ds(r0 + k * CHUNK, CHUNK)])
        return 0
    lax.fori_loop(0, ROWS_MAIN // CHUNK, zbody, 0)

    @pl.when(s < NS - 1)
    def _():
        pltpu.sync_copy(rows0.at[pl.ds(0, 56)],
                        acc_sh.at[pl.ds(r0 + ROWS_MAIN, 56)])

    @pl.when(s == NS - 1)
    def _():
        pltpu.sync_copy(rows0.at[pl.ds(0, 8)],
                        acc_sh.at[pl.ds(r0 + ROWS_MAIN, 8)])

    plsc.subcore_barrier()

    ebase0 = s * EPT

    def prep(cb, dloc, gsrc):
        @plsc.parallel_loop(0, CHUNK // 16, unroll=2)
        def _(q):
            d = dst_v[pl.ds(cb + q * 16, 16)]
            sv = src_v[pl.ds(cb + q * 16, 16)]
            inr = (d >= base_row) & (d < base_row + HALF)
            dloc[pl.ds(q * 16, 16)] = jnp.where(inr, d - base_row, -1)
            gsrc[pl.ds(q * 16, 16)] = jnp.where(inr, sv, -1)

    def gather(gsrc, rows, gsem):
        pltpu.async_copy(tbl_hbm.at[plsc.Indices(gsrc, ignored_value=-1)],
                         rows, gsem)

    def wait_gather(gsrc, rows, gsem):
        pltpu.make_async_copy(tbl_hbm.at[plsc.Indices(gsrc, ignored_value=-1)],
                              rows, gsem).wait()

    def scatter(rows, dloc, ssem):
        pltpu.async_copy(rows, acc_sh.at[plsc.Indices(dloc, ignored_value=-1)],
                         ssem, add=True)

    def wait_scatter(rows, dloc, ssem):
        pltpu.make_async_copy(rows,
                              acc_sh.at[plsc.Indices(dloc, ignored_value=-1)],
                              ssem).wait()

    def mul(rows, wbase):
        @plsc.parallel_loop(0, CHUNK, unroll=4)
        def _(j):
            widx = jnp.full((16,), wbase + j, jnp.int32)
            ws = plsc.load_gather(w_v, [widx])
            rows[j, 0:16] = rows[j, 0:16] * ws
            rows[j, 16:32] = rows[j, 16:32] * ws

    def super_body(g, _):
        eb = ebase0 + g * SUPER
        pltpu.sync_copy(src_hbm.at[pl.ds(eb, SUPER)], src_v)
        pltpu.sync_copy(dst_hbm.at[pl.ds(eb, SUPER)], dst_v)
        pltpu.sync_copy(w_hbm.at[pl.ds(eb, SUPER)], w_v)

        prep(0, dloc0, gsrc0)
        gather(gsrc0, rows0, gsem0)

        def pair(p, _):
            wait_gather(gsrc0, rows0, gsem0)
            mul(rows0, 2 * p * CHUNK)

            @pl.when(p > 0)
            def _():
                wait_scatter(rows1, dloc1, ssem1)
            prep((2 * p + 1) * CHUNK, dloc1, gsrc1)
            gather(gsrc1, rows1, gsem1)
            scatter(rows0, dloc0, ssem0)

            wait_gather(gsrc1, rows1, gsem1)
            mul(rows1, (2 * p + 1) * CHUNK)

            @pl.when(p < NCH // 2 - 1)
            def _():
                wait_scatter(rows0, dloc0, ssem0)
                prep((2 * p + 2) * CHUNK, dloc0, gsrc0)
                gather(gsrc0, rows0, gsem0)
            scatter(rows1, dloc1, ssem1)
            return 0
        lax.fori_loop(0, NCH // 2, pair, 0)

        wait_scatter(rows0, dloc0, ssem0)
        wait_scatter(rows1, dloc1, ssem1)
        return 0
    lax.fori_loop(0, NSUP, super_body, 0)

    plsc.subcore_barrier()
    pltpu.sync_copy(acc_sh.at[pl.ds(r0, ROWS_MAIN)],
                    out_hbm.at[pl.ds(base_row + r0, ROWS_MAIN)])

    @pl.when(s < NS - 1)
    def _():
        pltpu.sync_copy(acc_sh.at[pl.ds(r0 + ROWS_MAIN, 56)],
                        out_hbm.at[pl.ds(base_row + r0 + ROWS_MAIN, 56)])

    @pl.when(s == NS - 1)
    def _():
        pltpu.sync_copy(acc_sh.at[pl.ds(r0 + ROWS_MAIN, 8)],
                        out_hbm.at[pl.ds(base_row + r0 + ROWS_MAIN, 8)])


_sc_params = pltpu.CompilerParams(needs_layout_passes=False,
                                 use_tc_tiling_on_sc=False)

_layer = functools.partial(
    pl.kernel,
    out_type=jax.ShapeDtypeStruct((N, D), jnp.float32),
    mesh=_mesh,
    compiler_params=_sc_params,
    scratch_types=[
        pltpu.VMEM((SUPER,), jnp.int32),
        pltpu.VMEM((SUPER,), jnp.int32),
        pltpu.VMEM((SUPER,), jnp.float32),
        pltpu.VMEM((CHUNK,), jnp.int32),
        pltpu.VMEM((CHUNK,), jnp.int32),
        pltpu.VMEM((CHUNK,), jnp.int32),
        pltpu.VMEM((CHUNK,), jnp.int32),
        pltpu.VMEM((CHUNK, D), jnp.float32),
        pltpu.VMEM((CHUNK, D), jnp.float32),
        pltpu.VMEM_SHARED((HALF + 8, D), jnp.float32),
        pltpu.SemaphoreType.DMA,
        pltpu.SemaphoreType.DMA,
        pltpu.SemaphoreType.DMA,
        pltpu.SemaphoreType.DMA,
    ],
)(_layer_body)


def _sampler_body(e0, e1, e2, e3, idx_hbm, out_sum, out_e0,
                  idx_v, identw_v, rows_a, rows_b, sum_sh, sem):
    c = lax.axis_index("c")
    s = lax.axis_index("s")
    w = s * NC + c
    base = w * IPW
    srow = s * CHUNK

    def ibody(q, _):
        identw_v[pl.ds(q * 16, 16)] = lax.iota(jnp.int32, 16) + (q * 16 + srow)
        return 0
    lax.fori_loop(0, CHUNK // 16, ibody, 0)

    pltpu.sync_copy(idx_hbm.at[pl.ds(base, IPW)], idx_v)

    def sub(k, _):
        sl = idx_v.at[pl.ds(k * CHUNK, CHUNK)]
        pltpu.async_copy(e0.at[sl], rows_a, sem).wait()
        pltpu.sync_copy(rows_a, out_e0.at[pl.ds(base + k * CHUNK, CHUNK)])
        pltpu.sync_copy(rows_a, sum_sh.at[pl.ds(srow, CHUNK)])
        pltpu.async_copy(e1.at[sl], rows_b, sem).wait()
        pltpu.sync_copy(rows_b, sum_sh.at[identw_v], add=True)
        pltpu.async_copy(e2.at[sl], rows_b, sem).wait()
        pltpu.sync_copy(rows_b, sum_sh.at[identw_v], add=True)
        pltpu.async_copy(e3.at[sl], rows_b, sem).wait()
        pltpu.sync_copy(rows_b, sum_sh.at[identw_v], add=True)
        pltpu.sync_copy(sum_sh.at[pl.ds(srow, CHUNK)],
                        out_sum.at[pl.ds(base + k * CHUNK, CHUNK)])
        return 0
    lax.fori_loop(0, NSC, sub, 0)


_sampler = functools.partial(
    pl.kernel,
    out_type=(jax.ShapeDtypeStruct((SIDX, D), jnp.float32),
              jax.ShapeDtypeStruct((SIDX, D), jnp.float32)),
    mesh=_mesh,
    compiler_params=_sc_params,
    scratch_types=[
        pltpu.VMEM((IPW,), jnp.int32),
        pltpu.VMEM((CHUNK,), jnp.int32),
        pltpu.VMEM((CHUNK, D), jnp.float32),
        pltpu.VMEM((CHUNK, D), jnp.float32),
        pltpu.VMEM_SHARED((NS * CHUNK, D), jnp.float32),
        pltpu.SemaphoreType.DMA,
    ],
)(_sampler_body)


def _loss_body(sum_ref, e0_ref, out_ref):
    su = sum_ref[0:B, :] * 0.25
    sp = sum_ref[B:2 * B, :] * 0.25
    sn = sum_ref[2 * B:3 * B, :] * 0.25
    x = jnp.sum(su * sp, axis=1) - jnp.sum(su * sn, axis=1)
    ls = jnp.minimum(x, 0.0) - jnp.log(1.0 + jnp.exp(-jnp.abs(x)))
    loss = -jnp.mean(ls)
    u0 = e0_ref[0:B, :]
    p0 = e0_ref[B:2 * B, :]
    n0 = e0_ref[2 * B:3 * B, :]
    reg = jnp.mean(jnp.sum(u0 * u0 + p0 * p0 + n0 * n0, axis=1))
    out_ref[0] = loss
    out_ref[1] = reg


def _loss_tc(sum_rows, e0_rows):
    return pl.pallas_call(
        _loss_body,
        out_shape=jax.ShapeDtypeStruct((2,), jnp.float32),
        in_specs=[pl.BlockSpec(memory_space=pltpu.VMEM),
                  pl.BlockSpec(memory_space=pltpu.VMEM)],
        out_specs=pl.BlockSpec(memory_space=pltpu.SMEM),
    )(sum_rows, e0_rows)


def kernel(users, pos, neg, edge_index, edge_weight, user_emb, item_emb):
    e0 = jnp.concatenate([user_emb, item_emb], axis=0)
    pad = E_PAD - E
    src = jnp.pad(edge_index[0], (0, pad))
    dst = jnp.pad(edge_index[1], (0, pad), constant_values=-1)
    w = jnp.pad(edge_weight, (0, pad))
    e1 = _layer(src, dst, w, e0)
    e2 = _layer(src, dst, w, e1)
    e3 = _layer(src, dst, w, e2)
    idx = jnp.concatenate([users, pos + N_USERS, neg + N_USERS])
    sum_rows, e0_rows = _sampler(e0, e1, e2, e3, idx)
    return _loss_tc(sum_rows, e0_rows)


# speedup vs baseline: 18.1292x; 2.1769x over previous
import functools

import jax
import jax.numpy as jnp
from jax import lax
from jax.experimental import pallas as pl
from jax.experimental.pallas import tpu as pltpu
from jax.experimental.pallas import tpu_sc as plsc

N_USERS = 50000
M_ITEMS = 50000
D = 32
N = N_USERS + M_ITEMS
E = 1600000
B = 4096

NC = 2
NS = 16
HALF = N // NC
ROWS_PT = 3128
ROWS_MAIN = 3072
DUMP = HALF

CHUNK = 128
SUPER = 4096
NCH = SUPER // CHUNK
E_PAD = 1638400
EPT = E_PAD // NS
NSUP = EPT // SUPER

SIDX = 3 * B
IPW = SIDX // (NC * NS)
NSC = IPW // CHUNK

_mesh = plsc.VectorSubcoreMesh(core_axis_name="c", subcore_axis_name="s")


def _layer_body(src_hbm, dst_hbm, w_hbm, tbl_hbm, out_hbm,
                src_v, dst_v, w_v, dloc0, dloc1, dloc2, dloc3,
                gsrc0, gsrc1, gsrc2, gsrc3, rows0, rows1, rows2, rows3,
                acc_sh, gsem0, gsem1, gsem2, gsem3,
                ssem0, ssem1, ssem2, ssem3):
    c = lax.axis_index("c")
    s = lax.axis_index("s")
    base_row = c * HALF
    r0 = s * ROWS_PT

    zero16 = jnp.zeros((16,), jnp.float32)

    @plsc.parallel_loop(0, CHUNK, unroll=4)
    def _(j):
        rows0[j, 0:16] = zero16
        rows0[j, 16:32] = zero16

    def zbody(k, _):
        pltpu.sync_copy(rows0, acc_sh.at[pl.ds(r0 + k * CHUNK, CHUNK)])
        return 0
    lax.fori_loop(0, ROWS_MAIN // CHUNK, zbody, 0)

    @pl.when(s < NS - 1)
    def _():
        pltpu.sync_copy(rows0.at[pl.ds(0, 56)],
                        acc_sh.at[pl.ds(r0 + ROWS_MAIN, 56)])

    @pl.when(s == NS - 1)
    def _():
        pltpu.sync_copy(rows0.at[pl.ds(0, 8)],
                        acc_sh.at[pl.ds(r0 + ROWS_MAIN, 8)])

    plsc.subcore_barrier()

    ebase0 = s * EPT

    def prep(cb, dloc, gsrc):
        @plsc.parallel_loop(0, CHUNK // 16, unroll=2)
        def _(q):
            d = dst_v[pl.ds(cb + q * 16, 16)]
            sv = src_v[pl.ds(cb + q * 16, 16)]
            inr = (d >= base_row) & (d < base_row + HALF)
            dloc[pl.ds(q * 16, 16)] = jnp.where(inr, d - base_row, -1)
            gsrc[pl.ds(q * 16, 16)] = jnp.where(inr, sv, -1)

    def gather(gsrc, rows, gsem):
        pltpu.async_copy(tbl_hbm.at[plsc.Indices(gsrc, ignored_value=-1)],
                         rows, gsem)

    def wait_gather(gsrc, rows, gsem):
        pltpu.make_async_copy(tbl_hbm.at[plsc.Indices(gsrc, ignored_value=-1)],
                              rows, gsem).wait()

    def scatter(rows, dloc, ssem):
        pltpu.async_copy(rows, acc_sh.at[plsc.Indices(dloc, ignored_value=-1)],
                         ssem, add=True)

    def wait_scatter(rows, dloc, ssem):
        pltpu.make_async_copy(rows,
                              acc_sh.at[plsc.Indices(dloc, ignored_value=-1)],
                              ssem).wait()

    def mul(rows, wbase):
        @plsc.parallel_loop(0, CHUNK, unroll=4)
        def _(j):
            widx = jnp.full((16,), wbase + j, jnp.int32)
            ws = plsc.load_gather(w_v, [widx])
            rows[j, 0:16] = rows[j, 0:16] * ws
            rows[j, 16:32] = rows[j, 16:32] * ws

    def super_body(g, _):
        eb = ebase0 + g * SUPER
        pltpu.sync_copy(src_hbm.at[pl.ds(eb, SUPER)], src_v)
        pltpu.sync_copy(dst_hbm.at[pl.ds(eb, SUPER)], dst_v)
        pltpu.sync_copy(w_hbm.at[pl.ds(eb, SUPER)], w_v)

        dlocs = (dloc0, dloc1, dloc2, dloc3)
        gsrcs = (gsrc0, gsrc1, gsrc2, gsrc3)
        rowss = (rows0, rows1, rows2, rows3)
        gsems = (gsem0, gsem1, gsem2, gsem3)
        ssems = (ssem0, ssem1, ssem2, ssem3)

        for b in range(3):
            prep(b * CHUNK, dlocs[b], gsrcs[b])
            gather(gsrcs[b], rowss[b], gsems[b])

        def quad(p, _):
            for b in range(4):
                cb = (4 * p + b) * CHUNK
                wait_gather(gsrcs[b], rowss[b], gsems[b])
                mul(rowss[b], cb)
                scatter(rowss[b], dlocs[b], ssems[b])
                bn = (b + 3) % 4
                if b == 0:
                    @pl.when(p > 0)
                    def _():
                        wait_scatter(rowss[bn], dlocs[bn], ssems[bn])
                    prep(cb + 3 * CHUNK, dlocs[bn], gsrcs[bn])
                    gather(gsrcs[bn], rowss[bn], gsems[bn])
                else:
                    wait_scatter(rowss[bn], dlocs[bn], ssems[bn])

                    @pl.when(p < NCH // 4 - 1)
                    def _():
                        prep(cb + 3 * CHUNK, dlocs[bn], gsrcs[bn])
                        gather(gsrcs[bn], rowss[bn], gsems[bn])
            return 0
        lax.fori_loop(0, NCH // 4, quad, 0)

        wait_scatter(rows3, dloc3, ssem3)
        return 0
    lax.fori_loop(0, NSUP, super_body, 0)

    plsc.subcore_barrier()
    pltpu.sync_copy(acc_sh.at[pl.ds(r0, ROWS_MAIN)],
                    out_hbm.at[pl.ds(base_row + r0, ROWS_MAIN)])

    @pl.when(s < NS - 1)
    def _():
        pltpu.sync_copy(acc_sh.at[pl.ds(r0 + ROWS_MAIN, 56)],
                        out_hbm.at[pl.ds(base_row + r0 + ROWS_MAIN, 56)])

    @pl.when(s == NS - 1)
    def _():
        pltpu.sync_copy(acc_sh.at[pl.ds(r0 + ROWS_MAIN, 8)],
                        out_hbm.at[pl.ds(base_row + r0 + ROWS_MAIN, 8)])


_sc_params = pltpu.CompilerParams(needs_layout_passes=False,
                                 use_tc_tiling_on_sc=False)

_layer = functools.partial(
    pl.kernel,
    out_type=jax.ShapeDtypeStruct((N, D), jnp.float32),
    mesh=_mesh,
    compiler_params=_sc_params,
    scratch_types=[
        pltpu.VMEM((SUPER,), jnp.int32),
        pltpu.VMEM((SUPER,), jnp.int32),
        pltpu.VMEM((SUPER,), jnp.float32),
        pltpu.VMEM((CHUNK,), jnp.int32),
        pltpu.VMEM((CHUNK,), jnp.int32),
        pltpu.VMEM((CHUNK,), jnp.int32),
        pltpu.VMEM((CHUNK,), jnp.int32),
        pltpu.VMEM((CHUNK,), jnp.int32),
        pltpu.VMEM((CHUNK,), jnp.int32),
        pltpu.VMEM((CHUNK,), jnp.int32),
        pltpu.VMEM((CHUNK,), jnp.int32),
        pltpu.VMEM((CHUNK, D), jnp.float32),
        pltpu.VMEM((CHUNK, D), jnp.float32),
        pltpu.VMEM((CHUNK, D), jnp.float32),
        pltpu.VMEM((CHUNK, D), jnp.float32),
        pltpu.VMEM_SHARED((HALF + 8, D), jnp.float32),
        pltpu.SemaphoreType.DMA,
        pltpu.SemaphoreType.DMA,
        pltpu.SemaphoreType.DMA,
        pltpu.SemaphoreType.DMA,
        pltpu.SemaphoreType.DMA,
        pltpu.SemaphoreType.DMA,
        pltpu.SemaphoreType.DMA,
        pltpu.SemaphoreType.DMA,
    ],
)(_layer_body)


def _sampler_body(e0, e1, e2, e3, idx_hbm, out_sum, out_e0,
                  idx_v, identw_v, rows_a, rows_b, sum_sh, sem):
    c = lax.axis_index("c")
    s = lax.axis_index("s")
    w = s * NC + c
    base = w * IPW
    srow = s * CHUNK

    def ibody(q, _):
        identw_v[pl.ds(q * 16, 16)] = lax.iota(jnp.int32, 16) + (q * 16 + srow)
        return 0
    lax.fori_loop(0, CHUNK // 16, ibody, 0)

    pltpu.sync_copy(idx_hbm.at[pl.ds(base, IPW)], idx_v)

    def sub(k, _):
        sl = idx_v.at[pl.ds(k * CHUNK, CHUNK)]
        pltpu.async_copy(e0.at[sl], rows_a, sem).wait()
        pltpu.sync_copy(rows_a, out_e0.at[pl.ds(base + k * CHUNK, CHUNK)])
        pltpu.sync_copy(rows_a, sum_sh.at[pl.ds(srow, CHUNK)])
        pltpu.async_copy(e1.at[sl], rows_b, sem).wait()
        pltpu.sync_copy(rows_b, sum_sh.at[identw_v], add=True)
        pltpu.async_copy(e2.at[sl], rows_b, sem).wait()
        pltpu.sync_copy(rows_b, sum_sh.at[identw_v], add=True)
        pltpu.async_copy(e3.at[sl], rows_b, sem).wait()
        pltpu.sync_copy(rows_b, sum_sh.at[identw_v], add=True)
        pltpu.sync_copy(sum_sh.at[pl.ds(srow, CHUNK)],
                        out_sum.at[pl.ds(base + k * CHUNK, CHUNK)])
        return 0
    lax.fori_loop(0, NSC, sub, 0)


_sampler = functools.partial(
    pl.kernel,
    out_type=(jax.ShapeDtypeStruct((SIDX, D), jnp.float32),
              jax.ShapeDtypeStruct((SIDX, D), jnp.float32)),
    mesh=_mesh,
    compiler_params=_sc_params,
    scratch_types=[
        pltpu.VMEM((IPW,), jnp.int32),
        pltpu.VMEM((CHUNK,), jnp.int32),
        pltpu.VMEM((CHUNK, D), jnp.float32),
        pltpu.VMEM((CHUNK, D), jnp.float32),
        pltpu.VMEM_SHARED((NS * CHUNK, D), jnp.float32),
        pltpu.SemaphoreType.DMA,
    ],
)(_sampler_body)


def _loss_body(sum_ref, e0_ref, out_ref):
    su = sum_ref[0:B, :] * 0.25
    sp = sum_ref[B:2 * B, :] * 0.25
    sn = sum_ref[2 * B:3 * B, :] * 0.25
    x = jnp.sum(su * sp, axis=1) - jnp.sum(su * sn, axis=1)
    ls = jnp.minimum(x, 0.0) - jnp.log(1.0 + jnp.exp(-jnp.abs(x)))
    loss = -jnp.mean(ls)
    u0 = e0_ref[0:B, :]
    p0 = e0_ref[B:2 * B, :]
    n0 = e0_ref[2 * B:3 * B, :]
    reg = jnp.mean(jnp.sum(u0 * u0 + p0 * p0 + n0 * n0, axis=1))
    out_ref[0] = loss
    out_ref[1] = reg


def _loss_tc(sum_rows, e0_rows):
    return pl.pallas_call(
        _loss_body,
        out_shape=jax.ShapeDtypeStruct((2,), jnp.float32),
        in_specs=[pl.BlockSpec(memory_space=pltpu.VMEM),
                  pl.BlockSpec(memory_space=pltpu.VMEM)],
        out_specs=pl.BlockSpec(memory_space=pltpu.SMEM),
    )(sum_rows, e0_rows)


def kernel(users, pos, neg, edge_index, edge_weight, user_emb, item_emb):
    e0 = jnp.concatenate([user_emb, item_emb], axis=0)
    pad = E_PAD - E
    src = jnp.pad(edge_index[0], (0, pad))
    dst = jnp.pad(edge_index[1], (0, pad), constant_values=-1)
    w = jnp.pad(edge_weight, (0, pad))
    e1 = _layer(src, dst, w, e0)
    e2 = _layer(src, dst, w, e1)
    e3 = _layer(src, dst, w, e2)
    idx = jnp.concatenate([users, pos + N_USERS, neg + N_USERS])
    sum_rows, e0_rows = _sampler(e0, e1, e2, e3, idx)
    return _loss_tc(sum_rows, e0_rows)


# speedup vs baseline: 19.3954x; 1.0698x over previous
import functools

import jax
import jax.numpy as jnp
from jax import lax
from jax.experimental import pallas as pl
from jax.experimental.pallas import tpu as pltpu
from jax.experimental.pallas import tpu_sc as plsc

N_USERS = 50000
M_ITEMS = 50000
D = 32
N = N_USERS + M_ITEMS
E = 1600000
B = 4096

NC = 2
NS = 16
HALF = N // NC
ROWS_PT = 3128
ROWS_MAIN = 3072
DUMP = HALF

CHUNK = 128
SUPER = 2048
NCH = SUPER // CHUNK
E_PAD = 1638400
EPT = E_PAD // NS
NSUP = EPT // SUPER

SIDX = 3 * B
IPW = SIDX // (NC * NS)
NSC = IPW // CHUNK

_mesh = plsc.VectorSubcoreMesh(core_axis_name="c", subcore_axis_name="s")


def _layer_body(src_hbm, dst_hbm, w_hbm, tbl_hbm, out_hbm,
                src_a, dst_a, w_a, src_b, dst_b, w_b, stsem,
                dloc0, dloc1, dloc2, dloc3,
                gsrc0, gsrc1, gsrc2, gsrc3, rows0, rows1, rows2, rows3,
                acc_sh, gsem0, gsem1, gsem2, gsem3,
                ssem0, ssem1, ssem2, ssem3):
    c = lax.axis_index("c")
    s = lax.axis_index("s")
    base_row = c * HALF
    r0 = s * ROWS_PT

    zero16 = jnp.zeros((16,), jnp.float32)

    @plsc.parallel_loop(0, CHUNK, unroll=4)
    def _(j):
        rows0[j, 0:16] = zero16
        rows0[j, 16:32] = zero16

    def zbody(k, _):
        pltpu.async_copy(rows0, acc_sh.at[pl.ds(r0 + k * CHUNK, CHUNK)], gsem0)
        return 0
    lax.fori_loop(0, ROWS_MAIN // CHUNK, zbody, 0)

    def zwait(k, _):
        pltpu.make_async_copy(rows0, acc_sh.at[pl.ds(r0 + k * CHUNK, CHUNK)],
                              gsem0).wait()
        return 0
    lax.fori_loop(0, ROWS_MAIN // CHUNK, zwait, 0)

    @pl.when(s < NS - 1)
    def _():
        pltpu.sync_copy(rows0.at[pl.ds(0, 56)],
                        acc_sh.at[pl.ds(r0 + ROWS_MAIN, 56)])

    @pl.when(s == NS - 1)
    def _():
        pltpu.sync_copy(rows0.at[pl.ds(0, 8)],
                        acc_sh.at[pl.ds(r0 + ROWS_MAIN, 8)])

    plsc.subcore_barrier()

    ebase0 = s * EPT

    def prep(cb, dloc, gsrc, dv_ref, sv_ref):
        @plsc.parallel_loop(0, CHUNK // 16, unroll=2)
        def _(q):
            d = dv_ref[pl.ds(cb + q * 16, 16)]
            sv = sv_ref[pl.ds(cb + q * 16, 16)]
            inr = (d >= base_row) & (d < base_row + HALF)
            dloc[pl.ds(q * 16, 16)] = jnp.where(inr, d - base_row, -1)
            gsrc[pl.ds(q * 16, 16)] = jnp.where(inr, sv, -1)

    def gather(gsrc, rows, gsem):
        pltpu.async_copy(tbl_hbm.at[plsc.Indices(gsrc, ignored_value=-1)],
                         rows, gsem)

    def wait_gather(gsrc, rows, gsem):
        pltpu.make_async_copy(tbl_hbm.at[plsc.Indices(gsrc, ignored_value=-1)],
                              rows, gsem).wait()

    def scatter(rows, dloc, ssem):
        pltpu.async_copy(rows, acc_sh.at[plsc.Indices(dloc, ignored_value=-1)],
                         ssem, add=True)

    def wait_scatter(rows, dloc, ssem):
        pltpu.make_async_copy(rows,
                              acc_sh.at[plsc.Indices(dloc, ignored_value=-1)],
                              ssem).wait()

    def mul(rows, wbase, wv_ref):
        @plsc.parallel_loop(0, CHUNK, unroll=8)
        def _(j):
            widx = jnp.full((16,), wbase + j, jnp.int32)
            ws = plsc.load_gather(wv_ref, [widx])
            rows[j, 0:16] = rows[j, 0:16] * ws
            rows[j, 16:32] = rows[j, 16:32] * ws

    dlocs = (dloc0, dloc1, dloc2, dloc3)
    gsrcs = (gsrc0, gsrc1, gsrc2, gsrc3)
    rowss = (rows0, rows1, rows2, rows3)
    gsems = (gsem0, gsem1, gsem2, gsem3)
    ssems = (ssem0, ssem1, ssem2, ssem3)
    stage_a = (src_a, dst_a, w_a)
    stage_b = (src_b, dst_b, w_b)

    for ref, hbm in zip(stage_a, (src_hbm, dst_hbm, w_hbm)):
        pltpu.sync_copy(hbm.at[pl.ds(ebase0, SUPER)], ref)

    def run_super(g, cur, nxt):
        sv_ref, dv_ref, wv_ref = cur

        @pl.when(g + 1 < NSUP)
        def _():
            eb2 = ebase0 + (g + 1) * SUPER
            for ref, hbm in zip(nxt, (src_hbm, dst_hbm, w_hbm)):
                pltpu.async_copy(hbm.at[pl.ds(eb2, SUPER)], ref, stsem)

        for b in range(3):
            prep(b * CHUNK, dlocs[b], gsrcs[b], dv_ref, sv_ref)
            gather(gsrcs[b], rowss[b], gsems[b])

        def quad(p, _):
            for b in range(4):
                cb = (4 * p + b) * CHUNK
                wait_gather(gsrcs[b], rowss[b], gsems[b])
                mul(rowss[b], cb, wv_ref)
                scatter(rowss[b], dlocs[b], ssems[b])
                bn = (b + 3) % 4
                if b == 0:
                    @pl.when(p > 0)
                    def _():
                        wait_scatter(rowss[bn], dlocs[bn], ssems[bn])
                    prep(cb + 3 * CHUNK, dlocs[bn], gsrcs[bn], dv_ref, sv_ref)
                    gather(gsrcs[bn], rowss[bn], gsems[bn])
                else:
                    wait_scatter(rowss[bn], dlocs[bn], ssems[bn])

                    @pl.when(p < NCH // 4 - 1)
                    def _():
                        prep(cb + 3 * CHUNK, dlocs[bn], gsrcs[bn],
                             dv_ref, sv_ref)
                        gather(gsrcs[bn], rowss[bn], gsems[bn])
            return 0
        lax.fori_loop(0, NCH // 4, quad, 0)

        wait_scatter(rows3, dloc3, ssem3)

        @pl.when(g + 1 < NSUP)
        def _():
            eb2 = ebase0 + (g + 1) * SUPER
            for ref, hbm in zip(nxt, (src_hbm, dst_hbm, w_hbm)):
                pltpu.make_async_copy(hbm.at[pl.ds(eb2, SUPER)], ref,
                                      stsem).wait()

    def super_pair(h, _):
        run_super(2 * h, stage_a, stage_b)
        run_super(2 * h + 1, stage_b, stage_a)
        return 0
    lax.fori_loop(0, NSUP // 2, super_pair, 0)
    if NSUP % 2:
        run_super(NSUP - 1, stage_a, stage_b)

    plsc.subcore_barrier()
    pltpu.sync_copy(acc_sh.at[pl.ds(r0, ROWS_MAIN)],
                    out_hbm.at[pl.ds(base_row + r0, ROWS_MAIN)])

    @pl.when(s < NS - 1)
    def _():
        pltpu.sync_copy(acc_sh.at[pl.ds(r0 + ROWS_MAIN, 56)],
                        out_hbm.at[pl.ds(base_row + r0 + ROWS_MAIN, 56)])

    @pl.when(s == NS - 1)
    def _():
        pltpu.sync_copy(acc_sh.at[pl.ds(r0 + ROWS_MAIN, 8)],
                        out_hbm.at[pl.ds(base_row + r0 + ROWS_MAIN, 8)])


_sc_params = pltpu.CompilerParams(needs_layout_passes=False,
                                 use_tc_tiling_on_sc=False)

_layer = functools.partial(
    pl.kernel,
    out_type=jax.ShapeDtypeStruct((N, D), jnp.float32),
    mesh=_mesh,
    compiler_params=_sc_params,
    scratch_types=[
        pltpu.VMEM((SUPER,), jnp.int32),
        pltpu.VMEM((SUPER,), jnp.int32),
        pltpu.VMEM((SUPER,), jnp.float32),
        pltpu.VMEM((SUPER,), jnp.int32),
        pltpu.VMEM((SUPER,), jnp.int32),
        pltpu.VMEM((SUPER,), jnp.float32),
        pltpu.SemaphoreType.DMA,
        pltpu.VMEM((CHUNK,), jnp.int32),
        pltpu.VMEM((CHUNK,), jnp.int32),
        pltpu.VMEM((CHUNK,), jnp.int32),
        pltpu.VMEM((CHUNK,), jnp.int32),
        pltpu.VMEM((CHUNK,), jnp.int32),
        pltpu.VMEM((CHUNK,), jnp.int32),
        pltpu.VMEM((CHUNK,), jnp.int32),
        pltpu.VMEM((CHUNK,), jnp.int32),
        pltpu.VMEM((CHUNK, D), jnp.float32),
        pltpu.VMEM((CHUNK, D), jnp.float32),
        pltpu.VMEM((CHUNK, D), jnp.float32),
        pltpu.VMEM((CHUNK, D), jnp.float32),
        pltpu.VMEM_SHARED((HALF, D), jnp.float32),
        pltpu.SemaphoreType.DMA,
        pltpu.SemaphoreType.DMA,
        pltpu.SemaphoreType.DMA,
        pltpu.SemaphoreType.DMA,
        pltpu.SemaphoreType.DMA,
        pltpu.SemaphoreType.DMA,
        pltpu.SemaphoreType.DMA,
        pltpu.SemaphoreType.DMA,
    ],
)(_layer_body)


def _sampler_body(e0, e1, e2, e3, idx_hbm, out_sum, out_e0,
                  idx_v, identw_v, rows_a, rows_b, sum_sh, sem):
    c = lax.axis_index("c")
    s = lax.axis_index("s")
    w = s * NC + c
    base = w * IPW
    srow = s * CHUNK

    def ibody(q, _):
        identw_v[pl.ds(q * 16, 16)] = lax.iota(jnp.int32, 16) + (q * 16 + srow)
        return 0
    lax.fori_loop(0, CHUNK // 16, ibody, 0)

    pltpu.sync_copy(idx_hbm.at[pl.ds(base, IPW)], idx_v)

    def sub(k, _):
        sl = idx_v.at[pl.ds(k * CHUNK, CHUNK)]
        pltpu.async_copy(e0.at[sl], rows_a, sem).wait()
        pltpu.sync_copy(rows_a, out_e0.at[pl.ds(base + k * CHUNK, CHUNK)])
        pltpu.sync_copy(rows_a, sum_sh.at[pl.ds(srow, CHUNK)])
        pltpu.async_copy(e1.at[sl], rows_b, sem).wait()
        pltpu.sync_copy(rows_b, sum_sh.at[identw_v], add=True)
        pltpu.async_copy(e2.at[sl], rows_b, sem).wait()
        pltpu.sync_copy(rows_b, sum_sh.at[identw_v], add=True)
        pltpu.async_copy(e3.at[sl], rows_b, sem).wait()
        pltpu.sync_copy(rows_b, sum_sh.at[identw_v], add=True)
        pltpu.sync_copy(sum_sh.at[pl.ds(srow, CHUNK)],
                        out_sum.at[pl.ds(base + k * CHUNK, CHUNK)])
        return 0
    lax.fori_loop(0, NSC, sub, 0)


_sampler = functools.partial(
    pl.kernel,
    out_type=(jax.ShapeDtypeStruct((SIDX, D), jnp.float32),
              jax.ShapeDtypeStruct((SIDX, D), jnp.float32)),
    mesh=_mesh,
    compiler_params=_sc_params,
    scratch_types=[
        pltpu.VMEM((IPW,), jnp.int32),
        pltpu.VMEM((CHUNK,), jnp.int32),
        pltpu.VMEM((CHUNK, D), jnp.float32),
        pltpu.VMEM((CHUNK, D), jnp.float32),
        pltpu.VMEM_SHARED((NS * CHUNK, D), jnp.float32),
        pltpu.SemaphoreType.DMA,
    ],
)(_sampler_body)


def _loss_body(sum_ref, e0_ref, out_ref):
    su = sum_ref[0:B, :] * 0.25
    sp = sum_ref[B:2 * B, :] * 0.25
    sn = sum_ref[2 * B:3 * B, :] * 0.25
    x = jnp.sum(su * sp, axis=1) - jnp.sum(su * sn, axis=1)
    ls = jnp.minimum(x, 0.0) - jnp.log(1.0 + jnp.exp(-jnp.abs(x)))
    loss = -jnp.mean(ls)
    u0 = e0_ref[0:B, :]
    p0 = e0_ref[B:2 * B, :]
    n0 = e0_ref[2 * B:3 * B, :]
    reg = jnp.mean(jnp.sum(u0 * u0 + p0 * p0 + n0 * n0, axis=1))
    out_ref[0] = loss
    out_ref[1] = reg


def _loss_tc(sum_rows, e0_rows):
    return pl.pallas_call(
        _loss_body,
        out_shape=jax.ShapeDtypeStruct((2,), jnp.float32),
        in_specs=[pl.BlockSpec(memory_space=pltpu.VMEM),
                  pl.BlockSpec(memory_space=pltpu.VMEM)],
        out_specs=pl.BlockSpec(memory_space=pltpu.SMEM),
    )(sum_rows, e0_rows)


def kernel(users, pos, neg, edge_index, edge_weight, user_emb, item_emb):
    e0 = jnp.concatenate([user_emb, item_emb], axis=0)
    pad = E_PAD - E
    src = jnp.pad(edge_index[0], (0, pad))
    dst = jnp.pad(edge_index[1], (0, pad), constant_values=-1)
    w = jnp.pad(edge_weight, (0, pad))
    e1 = _layer(src, dst, w, e0)
    e2 = _layer(src, dst, w, e1)
    e3 = _layer(src, dst, w, e2)
    idx = jnp.concatenate([users, pos + N_USERS, neg + N_USERS])
    sum_rows, e0_rows = _sampler(e0, e1, e2, e3, idx)
    return _loss_tc(sum_rows, e0_rows)


# speedup vs baseline: 20.1239x; 1.0376x over previous
import functools

import jax
import jax.numpy as jnp
from jax import lax
from jax.experimental import pallas as pl
from jax.experimental.pallas import tpu as pltpu
from jax.experimental.pallas import tpu_sc as plsc

N_USERS = 50000
M_ITEMS = 50000
D = 32
N = N_USERS + M_ITEMS
E = 1600000
B = 4096

NC = 2
NS = 16
HALF = N // NC
ROWS_PT = 3128
ROWS_MAIN = 3072
DUMP = HALF

CHUNK = 128
SUPER = 2048
NCH = SUPER // CHUNK
E_PAD = 1638400
EPT = E_PAD // NS
NSUP = EPT // SUPER

SIDX = 3 * B
IPW = SIDX // (NC * NS)
NSC = IPW // CHUNK

_mesh = plsc.VectorSubcoreMesh(core_axis_name="c", subcore_axis_name="s")


def _layer_body(src_hbm, dst_hbm, w_hbm, tbl_hbm, out_hbm,
                src_a, dst_a, w_a, src_b, dst_b, w_b, stsem,
                dloc0, dloc1, dloc2, dloc3,
                gsrc0, gsrc1, gsrc2, gsrc3, rows0, rows1, rows2, rows3,
                acc_sh, gsem0, gsem1, gsem2, gsem3,
                ssem0, ssem1, ssem2, ssem3):
    c = lax.axis_index("c")
    s = lax.axis_index("s")
    base_row = c * HALF
    r0 = s * ROWS_PT

    zero16 = jnp.zeros((16,), jnp.float32)

    @plsc.parallel_loop(0, CHUNK, unroll=4)
    def _(j):
        rows0[j, 0:16] = zero16
        rows0[j, 16:32] = zero16

    def zbody(k, _):
        pltpu.async_copy(rows0, acc_sh.at[pl.ds(r0 + k * CHUNK, CHUNK)], gsem0)
        return 0
    lax.fori_loop(0, ROWS_MAIN // CHUNK, zbody, 0)

    def zwait(k, _):
        pltpu.make_async_copy(rows0, acc_sh.at[pl.ds(r0 + k * CHUNK, CHUNK)],
                              gsem0).wait()
        return 0
    lax.fori_loop(0, ROWS_MAIN // CHUNK, zwait, 0)

    @pl.when(s < NS - 1)
    def _():
        pltpu.sync_copy(rows0.at[pl.ds(0, 56)],
                        acc_sh.at[pl.ds(r0 + ROWS_MAIN, 56)])

    @pl.when(s == NS - 1)
    def _():
        pltpu.sync_copy(rows0.at[pl.ds(0, 8)],
                        acc_sh.at[pl.ds(r0 + ROWS_MAIN, 8)])

    plsc.subcore_barrier()

    ebase0 = s * EPT

    def prep(cb, dloc, gsrc, dv_ref, sv_ref):
        @plsc.parallel_loop(0, CHUNK // 16, unroll=2)
        def _(q):
            d = dv_ref[pl.ds(cb + q * 16, 16)]
            sv = sv_ref[pl.ds(cb + q * 16, 16)]
            inr = (d >= base_row) & (d < base_row + HALF)
            dloc[pl.ds(q * 16, 16)] = jnp.where(inr, d - base_row, -1)
            gsrc[pl.ds(q * 16, 16)] = jnp.where(inr, sv, -1)

    def gather(gsrc, rows, gsem):
        pltpu.async_copy(tbl_hbm.at[plsc.Indices(gsrc, ignored_value=-1)],
                         rows, gsem)

    def wait_gather(gsrc, rows, gsem):
        pltpu.make_async_copy(tbl_hbm.at[plsc.Indices(gsrc, ignored_value=-1)],
                              rows, gsem).wait()

    def scatter(rows, dloc, ssem):
        pltpu.async_copy(rows, acc_sh.at[plsc.Indices(dloc, ignored_value=-1)],
                         ssem, add=True)

    def wait_scatter(rows, dloc, ssem):
        pltpu.make_async_copy(rows,
                              acc_sh.at[plsc.Indices(dloc, ignored_value=-1)],
                              ssem).wait()

    def mul(rows, wbase, wv_ref):
        @plsc.parallel_loop(0, CHUNK // 16, unroll=2)
        def _(g):
            wvec = wv_ref[pl.ds(wbase + g * 16, 16)]
            for j in range(16):
                ws = jnp.full((16,), wvec[j])
                r = g * 16 + j
                rows[r, 0:16] = rows[r, 0:16] * ws
                rows[r, 16:32] = rows[r, 16:32] * ws

    dlocs = (dloc0, dloc1, dloc2, dloc3)
    gsrcs = (gsrc0, gsrc1, gsrc2, gsrc3)
    rowss = (rows0, rows1, rows2, rows3)
    gsems = (gsem0, gsem1, gsem2, gsem3)
    ssems = (ssem0, ssem1, ssem2, ssem3)
    stage_a = (src_a, dst_a, w_a)
    stage_b = (src_b, dst_b, w_b)

    for ref, hbm in zip(stage_a, (src_hbm, dst_hbm, w_hbm)):
        pltpu.sync_copy(hbm.at[pl.ds(ebase0, SUPER)], ref)

    def run_super(g, cur, nxt):
        sv_ref, dv_ref, wv_ref = cur

        @pl.when(g + 1 < NSUP)
        def _():
            eb2 = ebase0 + (g + 1) * SUPER
            for ref, hbm in zip(nxt, (src_hbm, dst_hbm, w_hbm)):
                pltpu.async_copy(hbm.at[pl.ds(eb2, SUPER)], ref, stsem)

        for b in range(3):
            prep(b * CHUNK, dlocs[b], gsrcs[b], dv_ref, sv_ref)
            gather(gsrcs[b], rowss[b], gsems[b])

        def quad(p, _):
            for b in range(4):
                cb = (4 * p + b) * CHUNK
                wait_gather(gsrcs[b], rowss[b], gsems[b])
                mul(rowss[b], cb, wv_ref)
                scatter(rowss[b], dlocs[b], ssems[b])
                bn = (b + 3) % 4
                if b == 0:
                    @pl.when(p > 0)
                    def _():
                        wait_scatter(rowss[bn], dlocs[bn], ssems[bn])
                    prep(cb + 3 * CHUNK, dlocs[bn], gsrcs[bn], dv_ref, sv_ref)
                    gather(gsrcs[bn], rowss[bn], gsems[bn])
                else:
                    wait_scatter(rowss[bn], dlocs[bn], ssems[bn])

                    @pl.when(p < NCH // 4 - 1)
                    def _():
                        prep(cb + 3 * CHUNK, dlocs[bn], gsrcs[bn],
                             dv_ref, sv_ref)
                        gather(gsrcs[bn], rowss[bn], gsems[bn])
            return 0
        lax.fori_loop(0, NCH // 4, quad, 0)

        wait_scatter(rows3, dloc3, ssem3)

        @pl.when(g + 1 < NSUP)
        def _():
            eb2 = ebase0 + (g + 1) * SUPER
            for ref, hbm in zip(nxt, (src_hbm, dst_hbm, w_hbm)):
                pltpu.make_async_copy(hbm.at[pl.ds(eb2, SUPER)], ref,
                                      stsem).wait()

    def super_pair(h, _):
        run_super(2 * h, stage_a, stage_b)
        run_super(2 * h + 1, stage_b, stage_a)
        return 0
    lax.fori_loop(0, NSUP // 2, super_pair, 0)
    if NSUP % 2:
        run_super(NSUP - 1, stage_a, stage_b)

    plsc.subcore_barrier()
    pltpu.sync_copy(acc_sh.at[pl.ds(r0, ROWS_MAIN)],
                    out_hbm.at[pl.ds(base_row + r0, ROWS_MAIN)])

    @pl.when(s < NS - 1)
    def _():
        pltpu.sync_copy(acc_sh.at[pl.ds(r0 + ROWS_MAIN, 56)],
                        out_hbm.at[pl.ds(base_row + r0 + ROWS_MAIN, 56)])

    @pl.when(s == NS - 1)
    def _():
        pltpu.sync_copy(acc_sh.at[pl.ds(r0 + ROWS_MAIN, 8)],
                        out_hbm.at[pl.ds(base_row + r0 + ROWS_MAIN, 8)])


_sc_params = pltpu.CompilerParams(needs_layout_passes=False,
                                 use_tc_tiling_on_sc=False)

_layer = functools.partial(
    pl.kernel,
    out_type=jax.ShapeDtypeStruct((N, D), jnp.float32),
    mesh=_mesh,
    compiler_params=_sc_params,
    scratch_types=[
        pltpu.VMEM((SUPER,), jnp.int32),
        pltpu.VMEM((SUPER,), jnp.int32),
        pltpu.VMEM((SUPER,), jnp.float32),
        pltpu.VMEM((SUPER,), jnp.int32),
        pltpu.VMEM((SUPER,), jnp.int32),
        pltpu.VMEM((SUPER,), jnp.float32),
        pltpu.SemaphoreType.DMA,
        pltpu.VMEM((CHUNK,), jnp.int32),
        pltpu.VMEM((CHUNK,), jnp.int32),
        pltpu.VMEM((CHUNK,), jnp.int32),
        pltpu.VMEM((CHUNK,), jnp.int32),
        pltpu.VMEM((CHUNK,), jnp.int32),
        pltpu.VMEM((CHUNK,), jnp.int32),
        pltpu.VMEM((CHUNK,), jnp.int32),
        pltpu.VMEM((CHUNK,), jnp.int32),
        pltpu.VMEM((CHUNK, D), jnp.float32),
        pltpu.VMEM((CHUNK, D), jnp.float32),
        pltpu.VMEM((CHUNK, D), jnp.float32),
        pltpu.VMEM((CHUNK, D), jnp.float32),
        pltpu.VMEM_SHARED((HALF, D), jnp.float32),
        pltpu.SemaphoreType.DMA,
        pltpu.SemaphoreType.DMA,
        pltpu.SemaphoreType.DMA,
        pltpu.SemaphoreType.DMA,
        pltpu.SemaphoreType.DMA,
        pltpu.SemaphoreType.DMA,
        pltpu.SemaphoreType.DMA,
        pltpu.SemaphoreType.DMA,
    ],
)(_layer_body)


def _sampler_body(e0, e1, e2, e3, idx_hbm, out_sum, out_e0,
                  idx_v, identw_v, rows_a, rows_b, sum_sh, sem):
    c = lax.axis_index("c")
    s = lax.axis_index("s")
    w = s * NC + c
    base = w * IPW
    srow = s * CHUNK

    def ibody(q, _):
        identw_v[pl.ds(q * 16, 16)] = lax.iota(jnp.int32, 16) + (q * 16 + srow)
        return 0
    lax.fori_loop(0, CHUNK // 16, ibody, 0)

    pltpu.sync_copy(idx_hbm.at[pl.ds(base, IPW)], idx_v)

    def sub(k, _):
        sl = idx_v.at[pl.ds(k * CHUNK, CHUNK)]
        pltpu.async_copy(e0.at[sl], rows_a, sem).wait()
        pltpu.sync_copy(rows_a, out_e0.at[pl.ds(base + k * CHUNK, CHUNK)])
        pltpu.sync_copy(rows_a, sum_sh.at[pl.ds(srow, CHUNK)])
        pltpu.async_copy(e1.at[sl], rows_b, sem).wait()
        pltpu.sync_copy(rows_b, sum_sh.at[identw_v], add=True)
        pltpu.async_copy(e2.at[sl], rows_b, sem).wait()
        pltpu.sync_copy(rows_b, sum_sh.at[identw_v], add=True)
        pltpu.async_copy(e3.at[sl], rows_b, sem).wait()
        pltpu.sync_copy(rows_b, sum_sh.at[identw_v], add=True)
        pltpu.sync_copy(sum_sh.at[pl.ds(srow, CHUNK)],
                        out_sum.at[pl.ds(base + k * CHUNK, CHUNK)])
        return 0
    lax.fori_loop(0, NSC, sub, 0)


_sampler = functools.partial(
    pl.kernel,
    out_type=(jax.ShapeDtypeStruct((SIDX, D), jnp.float32),
              jax.ShapeDtypeStruct((SIDX, D), jnp.float32)),
    mesh=_mesh,
    compiler_params=_sc_params,
    scratch_types=[
        pltpu.VMEM((IPW,), jnp.int32),
        pltpu.VMEM((CHUNK,), jnp.int32),
        pltpu.VMEM((CHUNK, D), jnp.float32),
        pltpu.VMEM((CHUNK, D), jnp.float32),
        pltpu.VMEM_SHARED((NS * CHUNK, D), jnp.float32),
        pltpu.SemaphoreType.DMA,
    ],
)(_sampler_body)


def _loss_body(sum_ref, e0_ref, out_ref):
    su = sum_ref[0:B, :] * 0.25
    sp = sum_ref[B:2 * B, :] * 0.25
    sn = sum_ref[2 * B:3 * B, :] * 0.25
    x = jnp.sum(su * sp, axis=1) - jnp.sum(su * sn, axis=1)
    ls = jnp.minimum(x, 0.0) - jnp.log(1.0 + jnp.exp(-jnp.abs(x)))
    loss = -jnp.mean(ls)
    u0 = e0_ref[0:B, :]
    p0 = e0_ref[B:2 * B, :]
    n0 = e0_ref[2 * B:3 * B, :]
    reg = jnp.mean(jnp.sum(u0 * u0 + p0 * p0 + n0 * n0, axis=1))
    out_ref[0] = loss
    out_ref[1] = reg


def _loss_tc(sum_rows, e0_rows):
    return pl.pallas_call(
        _loss_body,
        out_shape=jax.ShapeDtypeStruct((2,), jnp.float32),
        in_specs=[pl.BlockSpec(memory_space=pltpu.VMEM),
                  pl.BlockSpec(memory_space=pltpu.VMEM)],
        out_specs=pl.BlockSpec(memory_space=pltpu.SMEM),
    )(sum_rows, e0_rows)


def kernel(users, pos, neg, edge_index, edge_weight, user_emb, item_emb):
    e0 = jnp.concatenate([user_emb, item_emb], axis=0)
    pad = E_PAD - E
    src = jnp.pad(edge_index[0], (0, pad))
    dst = jnp.pad(edge_index[1], (0, pad), constant_values=-1)
    w = jnp.pad(edge_weight, (0, pad))
    e1 = _layer(src, dst, w, e0)
    e2 = _layer(src, dst, w, e1)
    e3 = _layer(src, dst, w, e2)
    idx = jnp.concatenate([users, pos + N_USERS, neg + N_USERS])
    sum_rows, e0_rows = _sampler(e0, e1, e2, e3, idx)
    return _loss_tc(sum_rows, e0_rows)
